# baseline (speedup 1.0000x reference)
import functools

import jax
import jax.numpy as jnp
from jax import lax
from jax.experimental import pallas as pl
from jax.experimental.pallas import tpu as pltpu
from jax.experimental.pallas import tpu_sc as plsc

BATCH = 4096
SEQ = 50
TOKEN_DIM = 64
NUM_CORES = 2
NUM_SUBCORES = 16
NUM_WORKERS = NUM_CORES * NUM_SUBCORES
B_PER_W = BATCH // NUM_WORKERS
ROWS_PER_WORKER = B_PER_W * SEQ
S_PER_CHUNK = 5
NUM_CHUNKS = SEQ // S_PER_CHUNK
CHUNK = S_PER_CHUNK * B_PER_W
DT = TOKEN_DIM // 8

_mesh = plsc.VectorSubcoreMesh(core_axis_name="c", subcore_axis_name="s")


@functools.partial(
    pl.kernel,
    out_type=jax.ShapeDtypeStruct((SEQ, DT, NUM_WORKERS, 8, B_PER_W), jnp.float32),
    mesh=_mesh,
    compiler_params=pltpu.CompilerParams(
        use_tc_tiling_on_sc=False, needs_layout_passes=False
    ),
    scratch_types=[
        pltpu.VMEM((ROWS_PER_WORKER,), jnp.int32),
        pltpu.VMEM((ROWS_PER_WORKER,), jnp.int32),
        pltpu.VMEM((CHUNK, TOKEN_DIM), jnp.float32),
        pltpu.VMEM((CHUNK, TOKEN_DIM), jnp.float32),
        pltpu.VMEM((TOKEN_DIM, B_PER_W), jnp.float32),
        pltpu.VMEM((TOKEN_DIM, B_PER_W), jnp.float32),
        pltpu.SemaphoreType.DMA,
        pltpu.SemaphoreType.DMA,
        pltpu.SemaphoreType.DMA,
        pltpu.SemaphoreType.DMA,
    ],
)
def _sc_gather(
    table_hbm, idx_hbm, out_hbm, idx_v, idx_t, g0, g1, t0, t1, gs0, gs1, os0, os1
):
    gbufs = (g0, g1)
    gsems = (gs0, gs1)
    tbufs = (t0, t1)
    osems = (os0, os1)

    wid = lax.axis_index("s") * NUM_CORES + lax.axis_index("c")
    pltpu.sync_copy(idx_hbm.at[pl.ds(wid * ROWS_PER_WORKER, ROWS_PER_WORKER)], idx_v)

    iota = lax.iota(jnp.int32, 16)

    @pl.loop(0, SEQ)
    def _build(s):
        for rq in range(B_PER_W // 16):
            vec = plsc.load_gather(idx_v, [(rq * 16 + iota) * SEQ + s])
            idx_t[pl.ds(s * B_PER_W + rq * 16, 16)] = vec

    def start_gather(c):
        b = c % 2
        return pltpu.async_copy(
            table_hbm.at[idx_t.at[pl.ds(c * CHUNK, CHUNK)]], gbufs[b], gsems[b]
        )

    g = [None, None]
    o = [None, None]
    g[0] = start_gather(0)
    sblk = 0
    for c in range(NUM_CHUNKS):
        gb = c % 2
        g[c % 2].wait()
        if c + 1 < NUM_CHUNKS:
            g[(c + 1) % 2] = start_gather(c + 1)
        gbuf = gbufs[gb]
        for s_local in range(S_PER_CHUNK):
            s = c * S_PER_CHUNK + s_local
            tb = sblk % 2
            tbuf = tbufs[tb]
            if o[tb] is not None:
                for cp in o[tb]:
                    cp.wait()

            base = s_local * B_PER_W

            @pl.loop(0, TOKEN_DIM)
            def _transpose(d, base=base, gbuf=gbuf, tbuf=tbuf):
                dvec = jnp.full((16,), d, jnp.int32)
                for rq in range(B_PER_W // 16):
                    v = plsc.load_gather(gbuf, [base + rq * 16 + iota, dvec])
                    tbuf[d, pl.ds(rq * 16, 16)] = v

            copies = []
            for dt in range(DT):
                copies.append(
                    pltpu.async_copy(
                        tbuf.at[pl.ds(dt * 8, 8)],
                        out_hbm.at[s, dt, wid],
                        osems[tb],
                    )
                )
            o[tb] = copies
            sblk += 1
    for tb in range(2):
        if o[tb] is not None:
            for cp in o[tb]:
                cp.wait()


def kernel(indices, embedding_weight):
    flat = indices.reshape(-1).astype(jnp.int32)
    out5 = _sc_gather(embedding_weight, flat)
    return jnp.transpose(out5, (2, 4, 0, 1, 3)).reshape(BATCH, SEQ, TOKEN_DIM)


# speedup vs baseline: 3.2569x; 3.2569x over previous
import functools

import jax
import jax.numpy as jnp
from jax import lax
from jax.experimental import pallas as pl
from jax.experimental.pallas import tpu as pltpu
from jax.experimental.pallas import tpu_sc as plsc

BATCH = 4096
SEQ = 50
TOKEN_DIM = 64
NUM_CORES = 2
NUM_SUBCORES = 16
NUM_WORKERS = NUM_CORES * NUM_SUBCORES
B_PER_W = BATCH // NUM_WORKERS
ROWS_PER_WORKER = B_PER_W * SEQ
S_PER_CHUNK = 5
NUM_CHUNKS = SEQ // S_PER_CHUNK
CHUNK = S_PER_CHUNK * B_PER_W
DT = TOKEN_DIM // 8
TPAD = B_PER_W + 1

_mesh = plsc.VectorSubcoreMesh(core_axis_name="c", subcore_axis_name="s")


@functools.partial(
    pl.kernel,
    out_type=jax.ShapeDtypeStruct((SEQ, DT, NUM_WORKERS, 8, B_PER_W), jnp.float32),
    mesh=_mesh,
    compiler_params=pltpu.CompilerParams(
        use_tc_tiling_on_sc=False, needs_layout_passes=False
    ),
    scratch_types=[
        pltpu.VMEM((ROWS_PER_WORKER,), jnp.int32),
        pltpu.VMEM((ROWS_PER_WORKER,), jnp.int32),
        pltpu.VMEM((CHUNK, TOKEN_DIM), jnp.float32),
        pltpu.VMEM((CHUNK, TOKEN_DIM), jnp.float32),
        pltpu.VMEM((DT, 8, TPAD), jnp.float32),
        pltpu.VMEM((DT, 8, TPAD), jnp.float32),
        pltpu.SemaphoreType.DMA,
        pltpu.SemaphoreType.DMA,
        pltpu.SemaphoreType.DMA,
        pltpu.SemaphoreType.DMA,
    ],
)
def _sc_gather(
    table_hbm, idx_hbm, out_hbm, idx_v, idx_t, g0, g1, t0, t1, gs0, gs1, os0, os1
):
    gbufs = (g0, g1)
    gsems = (gs0, gs1)
    tbufs = (t0, t1)
    osems = (os0, os1)

    wid = lax.axis_index("s") * NUM_CORES + lax.axis_index("c")
    pltpu.sync_copy(idx_hbm.at[pl.ds(wid * ROWS_PER_WORKER, ROWS_PER_WORKER)], idx_v)

    iota = lax.iota(jnp.int32, 16)

    @pl.loop(0, SEQ)
    def _build(s):
        for rq in range(B_PER_W // 16):
            vec = plsc.load_gather(idx_v, [(rq * 16 + iota) * SEQ + s])
            idx_t[pl.ds(s * B_PER_W + rq * 16, 16)] = vec

    def gather_desc(cc, b):
        return pltpu.make_async_copy(
            table_hbm.at[idx_t.at[pl.ds(cc * CHUNK, CHUNK)]], gbufs[b], gsems[b]
        )

    def tdrain(tb):
        for _ in range(DT):
            pltpu.make_async_copy(
                out_hbm.at[0, 0, 0],
                tbufs[tb].at[0, :, pl.ds(0, B_PER_W)],
                osems[tb],
            ).wait()

    gather_desc(0, 0).start()
    gather_desc(1, 1).start()

    @pl.loop(0, NUM_CHUNKS, step=2)
    def _chunks(c):
        for b in range(2):
            cc = c + b
            gather_desc(cc, b).wait()
            gbuf = gbufs[b]
            for s_local in range(S_PER_CHUNK):
                s = cc * S_PER_CHUNK + s_local
                tb = s_local % 2
                tbuf = tbufs[tb]
                not_first = (cc > 0) | (s_local >= 2)

                @pl.when(not_first)
                def _drain(tb=tb):
                    tdrain(tb)

                base = s_local * B_PER_W

                @plsc.parallel_loop(0, B_PER_W, unroll=2)
                def _transpose(ri, base=base, gbuf=gbuf, tbuf=tbuf):
                    rvec = jnp.full((16,), ri, jnp.int32)
                    for q in range(TOKEN_DIM // 16):
                        rows = q * 16 + iota
                        v = gbuf[base + ri, pl.ds(q * 16, 16)]
                        plsc.store_scatter(tbuf, [rows // 8, rows % 8, rvec], v)

                for dt in range(DT):
                    pltpu.make_async_copy(
                        tbuf.at[dt, :, pl.ds(0, B_PER_W)],
                        out_hbm.at[s, dt, wid],
                        osems[tb],
                    ).start()

            @pl.when(cc + 2 < NUM_CHUNKS)
            def _next(cc=cc, b=b):
                gather_desc(cc + 2, b).start()

    tdrain(0)
    tdrain(1)


def kernel(indices, embedding_weight):
    flat = indices.reshape(-1).astype(jnp.int32)
    out5 = _sc_gather(embedding_weight, flat)
    return jnp.transpose(out5, (2, 4, 0, 1, 3)).reshape(BATCH, SEQ, TOKEN_DIM)


# speedup vs baseline: 3.2633x; 1.0020x over previous
import functools

import jax
import jax.numpy as jnp
from jax import lax
from jax.experimental import pallas as pl
from jax.experimental.pallas import tpu as pltpu
from jax.experimental.pallas import tpu_sc as plsc

BATCH = 4096
SEQ = 50
TOKEN_DIM = 64
NUM_CORES = 2
NUM_SUBCORES = 16
NUM_WORKERS = NUM_CORES * NUM_SUBCORES
B_PER_W = BATCH // NUM_WORKERS
ROWS_PER_WORKER = B_PER_W * SEQ
S_PER_CHUNK = 5
NUM_CHUNKS = SEQ // S_PER_CHUNK
CHUNK = S_PER_CHUNK * B_PER_W
DT = TOKEN_DIM // 8
TPAD = B_PER_W + 1

_mesh = plsc.VectorSubcoreMesh(core_axis_name="c", subcore_axis_name="s")


@functools.partial(
    pl.kernel,
    out_type=jax.ShapeDtypeStruct((SEQ, DT, NUM_WORKERS, 8, B_PER_W), jnp.float32),
    mesh=_mesh,
    compiler_params=pltpu.CompilerParams(
        use_tc_tiling_on_sc=False, needs_layout_passes=False
    ),
    scratch_types=[
        pltpu.VMEM((ROWS_PER_WORKER,), jnp.int32),
        pltpu.VMEM((ROWS_PER_WORKER,), jnp.int32),
        pltpu.VMEM((CHUNK, TOKEN_DIM), jnp.float32),
        pltpu.VMEM((CHUNK, TOKEN_DIM), jnp.float32),
        pltpu.VMEM((DT, 8, TPAD), jnp.float32),
        pltpu.VMEM((DT, 8, TPAD), jnp.float32),
        pltpu.SemaphoreType.DMA,
        pltpu.SemaphoreType.DMA,
        pltpu.SemaphoreType.DMA,
        pltpu.SemaphoreType.DMA,
    ],
)
def _sc_gather(
    table_hbm, idx_hbm, out_hbm, idx_v, idx_t, g0, g1, t0, t1, gs0, gs1, os0, os1
):
    gbufs = (g0, g1)
    gsems = (gs0, gs1)
    tbufs = (t0, t1)
    osems = (os0, os1)

    wid = lax.axis_index("s") * NUM_CORES + lax.axis_index("c")
    pltpu.sync_copy(idx_hbm.at[pl.ds(wid * ROWS_PER_WORKER, ROWS_PER_WORKER)], idx_v)

    iota = lax.iota(jnp.int32, 16)

    @pl.loop(0, SEQ)
    def _build(s):
        for rq in range(B_PER_W // 16):
            vec = plsc.load_gather(idx_v, [(rq * 16 + iota) * SEQ + s])
            idx_t[pl.ds(s * B_PER_W + rq * 16, 16)] = vec

    def gather_desc(cc, b):
        return pltpu.make_async_copy(
            table_hbm.at[idx_t.at[pl.ds(cc * CHUNK, CHUNK)]], gbufs[b], gsems[b]
        )

    def tdrain(tb):
        for _ in range(DT):
            pltpu.make_async_copy(
                out_hbm.at[0, 0, 0],
                tbufs[tb].at[0, :, pl.ds(0, B_PER_W)],
                osems[tb],
            ).wait()

    gather_desc(0, 0).start()
    gather_desc(1, 1).start()

    @pl.loop(0, NUM_CHUNKS, step=2)
    def _chunks(c):
        for b in range(2):
            cc = c + b
            gather_desc(cc, b).wait()
            gbuf = gbufs[b]
            for s_local in range(S_PER_CHUNK):
                s = cc * S_PER_CHUNK + s_local
                tb = s_local % 2
                tbuf = tbufs[tb]
                not_first = (cc > 0) | (s_local >= 2)

                @pl.when(not_first)
                def _drain(tb=tb):
                    tdrain(tb)

                base = s_local * B_PER_W

                @plsc.parallel_loop(0, B_PER_W, unroll=2)
                def _transpose(ri, base=base, gbuf=gbuf, tbuf=tbuf):
                    rvec = jnp.full((16,), ri, jnp.int32)
                    for q in range(TOKEN_DIM // 16):
                        rows = q * 16 + iota
                        v = gbuf[base + ri, pl.ds(q * 16, 16)]
                        plsc.store_scatter(tbuf, [rows // 8, rows % 8, rvec], v)

                for dt in range(DT):
                    pltpu.make_async_copy(
                        tbuf.at[dt, :, pl.ds(0, B_PER_W)],
                        out_hbm.at[s, dt, wid],
                        osems[tb],
                    ).start()

            @pl.when(cc + 2 < NUM_CHUNKS)
            def _next(cc=cc, b=b):
                gather_desc(cc + 2, b).start()

    tdrain(0)
    tdrain(1)


def kernel(indices, embedding_weight):
    flat = (indices % jnp.int32(100000)).reshape(-1).astype(jnp.int32)
    out5 = _sc_gather(embedding_weight, flat)
    return jnp.transpose(out5, (2, 4, 0, 1, 3)).reshape(BATCH, SEQ, TOKEN_DIM)
